# baseline (speedup 1.0000x reference)
import functools
import jax
import jax.numpy as jnp
from jax import lax
from jax.experimental import pallas as pl
from jax.experimental.pallas import tpu as pltpu
from jax.experimental.pallas import tpu_sc as plsc

_W = 1024
_RPW = 64
_PCHUNK = 8192


def kernel(points, subject_garment_id, features):
    b, c, n = points.shape
    s, l = features.shape
    rows = c + l
    nw = 32
    mesh = plsc.VectorSubcoreMesh(core_axis_name="c", subcore_axis_name="s")
    n_chunks = n // _W

    @functools.partial(
        pl.kernel,
        mesh=mesh,
        out_type=jax.ShapeDtypeStruct((b, rows, n), jnp.float32),
        scratch_types=[
            pltpu.VMEM((8,), jnp.int32),
            pltpu.VMEM((s, l), jnp.float32),
            pltpu.VMEM((_RPW, _W), jnp.float32),
            pltpu.VMEM((_PCHUNK,), jnp.float32),
            pltpu.SemaphoreType.DMA,
        ],
        compiler_params=pltpu.CompilerParams(
            use_tc_tiling_on_sc=False, needs_layout_passes=False
        ),
    )
    def sck(pts_hbm, sid_hbm, feat_hbm, out_hbm, sidv, featv, buf, pbuf, sem):
        cid = lax.axis_index("c")
        scid = lax.axis_index("s")
        wid = scid * 2 + cid
        bi = wid // 4
        l0 = (wid % 4) * _RPW

        pltpu.sync_copy(sid_hbm, sidv)
        pltpu.sync_copy(feat_hbm, featv)

        sidvec = plsc.load_gather(sidv, [jnp.full((16,), bi, jnp.int32)])

        def fill_row(j, _):
            vj = plsc.load_gather(
                featv, [sidvec, jnp.full((16,), l0 + j, jnp.int32)]
            )

            def fill_col(k, _):
                buf[j, pl.ds(k * 16, 16)] = vj
                return 0

            return lax.fori_loop(0, _W // 16, fill_col, 0)

        lax.fori_loop(0, _RPW, fill_row, 0)

        copies = []
        for k in range(n_chunks):
            copies.append(
                pltpu.make_async_copy(
                    buf, out_hbm.at[bi, pl.ds(c + l0, _RPW), pl.ds(k * _W, _W)], sem
                )
            )
        for cp in copies:
            cp.start()
        for cp in copies:
            cp.wait()

        @pl.when(wid < b * c)
        def _():
            bp = wid // c
            ci = wid % c
            for j in range(n // _PCHUNK):
                pltpu.sync_copy(
                    pts_hbm.at[bp, ci, pl.ds(j * _PCHUNK, _PCHUNK)], pbuf
                )
                pltpu.sync_copy(
                    pbuf, out_hbm.at[bp, ci, pl.ds(j * _PCHUNK, _PCHUNK)]
                )

    return sck(points, subject_garment_id.astype(jnp.int32), features)


# speedup vs baseline: 1.0079x; 1.0079x over previous
import functools
import jax
import jax.numpy as jnp
from jax import lax
from jax.experimental import pallas as pl
from jax.experimental.pallas import tpu as pltpu
from jax.experimental.pallas import tpu_sc as plsc

_RPW = 64
_SEED = 4096
_PCHUNK = 8192


def kernel(points, subject_garment_id, features):
    b, c, n = points.shape
    s, l = features.shape
    rows = c + l
    mesh = plsc.VectorSubcoreMesh(core_axis_name="c", subcore_axis_name="s")
    n_rep = n // _SEED

    @functools.partial(
        pl.kernel,
        mesh=mesh,
        out_type=jax.ShapeDtypeStruct((b, rows, n), jnp.float32),
        scratch_types=[
            pltpu.VMEM((8,), jnp.int32),
            pltpu.VMEM((s, l), jnp.float32),
            pltpu.VMEM((_SEED,), jnp.float32),
            pltpu.VMEM((_SEED,), jnp.float32),
            pltpu.VMEM((_PCHUNK,), jnp.float32),
            pltpu.SemaphoreType.DMA,
            pltpu.SemaphoreType.DMA,
        ],
        compiler_params=pltpu.CompilerParams(
            use_tc_tiling_on_sc=False, needs_layout_passes=False
        ),
    )
    def sck(pts_hbm, sid_hbm, feat_hbm, out_hbm, sidv, featv, seed0, seed1,
            pbuf, sem0, sem1):
        cid = lax.axis_index("c")
        scid = lax.axis_index("s")
        wid = scid * 2 + cid
        bi = wid // 4
        l0 = (wid % 4) * _RPW

        pltpu.sync_copy(sid_hbm, sidv)
        pltpu.sync_copy(feat_hbm, featv)

        sidvec = plsc.load_gather(sidv, [jnp.full((16,), bi, jnp.int32)])

        bufs = (seed0, seed1)
        sems = (sem0, sem1)
        pending = [[], []]
        for j in range(_RPW):
            par = j % 2
            buf = bufs[par]
            li = l0 + j
            vj = plsc.load_gather(
                featv, [sidvec, jnp.full((16,), li, jnp.int32)]
            )
            for cp in pending[par]:
                cp.wait()
            pending[par] = []

            def fill(k, _):
                for u in range(8):
                    buf[pl.ds(k * 128 + u * 16, 16)] = vj
                return 0

            lax.fori_loop(0, _SEED // 128, fill, 0)
            for k in range(n_rep):
                cp = pltpu.make_async_copy(
                    buf, out_hbm.at[bi, c + li, pl.ds(k * _SEED, _SEED)],
                    sems[par],
                )
                cp.start()
                pending[par].append(cp)
        for par in (0, 1):
            for cp in pending[par]:
                cp.wait()

        @pl.when(wid < b * c)
        def _():
            bp = wid // c
            ci = wid % c
            for j in range(n // _PCHUNK):
                pltpu.sync_copy(
                    pts_hbm.at[bp, ci, pl.ds(j * _PCHUNK, _PCHUNK)], pbuf
                )
                pltpu.sync_copy(
                    pbuf, out_hbm.at[bp, ci, pl.ds(j * _PCHUNK, _PCHUNK)]
                )

    return sck(points, subject_garment_id.astype(jnp.int32), features)


# speedup vs baseline: 5.5550x; 5.5112x over previous
import jax
import jax.numpy as jnp
from jax.experimental import pallas as pl
from jax.experimental.pallas import tpu as pltpu

_BN = 16384


def _body(sid_ref, pts_hbm, feat_ref, out_hbm, t0, t1, pstage,
          s0, s1, sp):
    b, rows, n = out_hbm.shape
    c = pts_hbm.shape[1]
    nk = n // _BN
    tiles = (t0, t1)
    tsems = (s0, s1)

    pin = pltpu.make_async_copy(pts_hbm, pstage, sp)
    pin.start()

    big = [[], []]
    for bi in range(b):
        par = bi % 2
        for cp in big[par]:
            cp.wait()
        big[par] = []
        lat = feat_ref[sid_ref[bi]]
        tiles[par][...] = jnp.broadcast_to(lat, (rows, _BN))
        for k in range(nk):
            cp = pltpu.make_async_copy(
                tiles[par], out_hbm.at[bi, :, pl.ds(k * _BN, _BN)], tsems[par]
            )
            cp.start()
            big[par].append(cp)
    for par in (0, 1):
        for cp in big[par]:
            cp.wait()

    pin.wait()
    pout = pltpu.make_async_copy(pstage, out_hbm.at[:, pl.ds(0, c), :], sp)
    pout.start()
    pout.wait()


def kernel(points, subject_garment_id, features):
    b, c, n = points.shape
    s, l = features.shape
    rows = c + l
    feats_pad = jnp.concatenate(
        [jnp.zeros((s, c), jnp.float32), features], axis=1
    ).reshape(s, rows, 1)

    return pl.pallas_call(
        _body,
        grid_spec=pltpu.PrefetchScalarGridSpec(
            num_scalar_prefetch=1,
            grid=(1,),
            in_specs=[
                pl.BlockSpec(memory_space=pl.ANY),
                pl.BlockSpec((s, rows, 1), lambda i, sid: (0, 0, 0)),
            ],
            out_specs=pl.BlockSpec(memory_space=pl.ANY),
            scratch_shapes=[
                pltpu.VMEM((rows, _BN), jnp.float32),
                pltpu.VMEM((rows, _BN), jnp.float32),
                pltpu.VMEM((b, c, n), jnp.float32),
                pltpu.SemaphoreType.DMA,
                pltpu.SemaphoreType.DMA,
                pltpu.SemaphoreType.DMA,
            ],
        ),
        out_shape=jax.ShapeDtypeStruct((b, rows, n), jnp.float32),
        compiler_params=pltpu.CompilerParams(vmem_limit_bytes=100 * 1024 * 1024),
    )(subject_garment_id, points, feats_pad)


# speedup vs baseline: 5.5872x; 1.0058x over previous
import jax
import jax.numpy as jnp
from jax.experimental import pallas as pl
from jax.experimental.pallas import tpu as pltpu

_BN = 4096


def _body(sid_ref, pts_ref, feat_ref, out_ref):
    c = pts_ref.shape[1]
    rows, bn = out_ref.shape[1], out_ref.shape[2]
    out_ref[0] = jnp.broadcast_to(feat_ref[0], (rows, bn))
    out_ref[0, :c, :] = pts_ref[0]


def kernel(points, subject_garment_id, features):
    b, c, n = points.shape
    s, l = features.shape
    grid = (b, n // _BN)
    feats_pad = jnp.concatenate(
        [jnp.zeros((s, c), jnp.float32), features], axis=1
    ).reshape(s, c + l, 1)

    return pl.pallas_call(
        _body,
        grid_spec=pltpu.PrefetchScalarGridSpec(
            num_scalar_prefetch=1,
            grid=grid,
            in_specs=[
                pl.BlockSpec((1, c, _BN), lambda bi, ni, sid: (bi, 0, ni)),
                pl.BlockSpec((1, c + l, 1), lambda bi, ni, sid: (sid[bi], 0, 0)),
            ],
            out_specs=pl.BlockSpec(
                (1, c + l, _BN),
                lambda bi, ni, sid: (bi, 0, ni),
            ),
        ),
        out_shape=jax.ShapeDtypeStruct((b, c + l, n), jnp.float32),
    )(subject_garment_id, points, feats_pad)
